# Initial kernel scaffold; baseline (speedup 1.0000x reference)
import jax
import jax.numpy as jnp
from jax.experimental import pallas as pl


def kernel(x, edge_index, W, b):
    raise NotImplementedError("write your pallas kernel here")


# speedup vs baseline: 31.4098x; 31.4098x over previous
import functools

import jax
import jax.numpy as jnp
from jax import lax
from jax.experimental import pallas as pl
from jax.experimental.pallas import tpu as pltpu
from jax.experimental.pallas import tpu_sc as plsc

NC = 2
NS = 16
NW = NC * NS
CHUNK = 128


def _deg_body(npad, nch, dst3_hbm, degp_hbm, dst_v, ones_v, zero_v, deg_sh):
    c = lax.axis_index("c")
    s = lax.axis_index("s")
    wid = s * NC + c
    rpt = npad // NS

    for i in range(rpt // 16):
        zero_v[pl.ds(i * 16, 16)] = jnp.zeros((16,), jnp.float32)
    pltpu.sync_copy(zero_v, deg_sh.at[pl.ds(s * rpt, rpt)])

    for i in range(CHUNK // 16):
        ones_v[pl.ds(i * 16, 16)] = jnp.ones((16,), jnp.float32)
    pltpu.sync_copy(dst3_hbm.at[wid], dst_v)
    plsc.subcore_barrier()

    def step(j, _):
        pltpu.sync_copy(ones_v, deg_sh.at[dst_v.at[j]], add=True)
        return _

    lax.fori_loop(0, nch, step, 0)
    plsc.subcore_barrier()
    pltpu.sync_copy(deg_sh.at[pl.ds(s * rpt, rpt)],
                    degp_hbm.at[c].at[pl.ds(s * rpt, rpt)])


def _scat_body(npad, nch, d, y_hbm, src3_hbm, dst3_hbm, outp_hbm,
               src_v, dst_v, rows_v, accum_sh):
    c = lax.axis_index("c")
    s = lax.axis_index("s")
    wid = s * NC + c
    rpt = npad // NS

    pltpu.sync_copy(y_hbm.at[pl.ds(s * rpt, rpt)],
                    accum_sh.at[pl.ds(s * rpt, rpt)])
    pltpu.sync_copy(src3_hbm.at[wid], src_v)
    pltpu.sync_copy(dst3_hbm.at[wid], dst_v)
    plsc.subcore_barrier()

    def step(j, _):
        pltpu.sync_copy(y_hbm.at[src_v.at[j]], rows_v)
        pltpu.sync_copy(rows_v, accum_sh.at[dst_v.at[j]], add=True)
        return _

    lax.fori_loop(0, nch, step, 0)
    plsc.subcore_barrier()
    pltpu.sync_copy(accum_sh.at[pl.ds(s * rpt, rpt)],
                    outp_hbm.at[c].at[pl.ds(s * rpt, rpt)])


def _mm_body(x_ref, w_ref, degp_ref, y_ref, dinv_ref):
    deg = degp_ref[0] + degp_ref[1] + 1.0
    dinv = lax.rsqrt(deg)
    xw = jnp.dot(x_ref[...], w_ref[...], preferred_element_type=jnp.float32)
    y_ref[...] = xw * dinv
    dinv_ref[...] = dinv


def _fin_body(outp_ref, y_ref, dinv_ref, b_ref, out_ref):
    acc = outp_ref[0] + outp_ref[1] - y_ref[...]
    out_ref[...] = jnp.maximum(acc * dinv_ref[...] + b_ref[...], 0.0)


def kernel(x, edge_index, W, b):
    N, D = x.shape
    E = edge_index.shape[1]
    NPAD = ((N + NS * 16 - 1) // (NS * 16)) * (NS * 16)
    NPAD = max(NPAD, ((N + 127) // 128) * 128)
    nch = -(-E // (NW * CHUNK))
    epad = NW * CHUNK * nch - E

    src = edge_index[0].astype(jnp.int32)
    dst = edge_index[1].astype(jnp.int32)
    pad_idx = N + (jnp.arange(epad, dtype=jnp.int32) % (NPAD - N))
    src3 = jnp.concatenate([src, pad_idx]).reshape(NW, nch, CHUNK)
    dst3 = jnp.concatenate([dst, pad_idx]).reshape(NW, nch, CHUNK)
    x_pad = jnp.concatenate([x, jnp.zeros((NPAD - N, D), x.dtype)])

    mesh = plsc.VectorSubcoreMesh(core_axis_name="c", subcore_axis_name="s",
                                  num_cores=NC, num_subcores=NS)

    degp = pl.kernel(
        functools.partial(_deg_body, NPAD, nch),
        out_type=jax.ShapeDtypeStruct((NC, NPAD), jnp.float32),
        mesh=mesh,
        scratch_types=[
            pltpu.VMEM((nch, CHUNK), jnp.int32),
            pltpu.VMEM((CHUNK,), jnp.float32),
            pltpu.VMEM((NPAD // NS,), jnp.float32),
            pltpu.VMEM_SHARED((NPAD,), jnp.float32),
        ],
    )(dst3)

    BR = NPAD // 8
    y_pad, dinv = pl.pallas_call(
        _mm_body,
        grid=(8,),
        in_specs=[
            pl.BlockSpec((BR, D), lambda i: (i, 0)),
            pl.BlockSpec((D, D), lambda i: (0, 0)),
            pl.BlockSpec((NC, BR, 1), lambda i: (0, i, 0)),
        ],
        out_specs=[
            pl.BlockSpec((BR, D), lambda i: (i, 0)),
            pl.BlockSpec((BR, 1), lambda i: (i, 0)),
        ],
        out_shape=[
            jax.ShapeDtypeStruct((NPAD, D), jnp.float32),
            jax.ShapeDtypeStruct((NPAD, 1), jnp.float32),
        ],
    )(x_pad, W, degp.reshape(NC, NPAD, 1))

    outp = pl.kernel(
        functools.partial(_scat_body, NPAD, nch, D),
        out_type=jax.ShapeDtypeStruct((NC, NPAD, D), jnp.float32),
        mesh=mesh,
        scratch_types=[
            pltpu.VMEM((nch, CHUNK), jnp.int32),
            pltpu.VMEM((nch, CHUNK), jnp.int32),
            pltpu.VMEM((CHUNK, D), jnp.float32),
            pltpu.VMEM_SHARED((NPAD, D), jnp.float32),
        ],
    )(y_pad, src3, dst3)

    RB = 2000
    out = pl.pallas_call(
        _fin_body,
        grid=(N // RB,),
        in_specs=[
            pl.BlockSpec((NC, RB, D), lambda i: (0, i, 0)),
            pl.BlockSpec((RB, D), lambda i: (i, 0)),
            pl.BlockSpec((RB, 1), lambda i: (i, 0)),
            pl.BlockSpec((1, D), lambda i: (0, 0)),
        ],
        out_specs=pl.BlockSpec((RB, D), lambda i: (i, 0)),
        out_shape=jax.ShapeDtypeStruct((N, D), jnp.float32),
    )(outp, y_pad, dinv, b.reshape(1, D))
    return out


# speedup vs baseline: 42.7421x; 1.3608x over previous
import functools

import jax
import jax.numpy as jnp
from jax import lax
from jax.experimental import pallas as pl
from jax.experimental.pallas import tpu as pltpu
from jax.experimental.pallas import tpu_sc as plsc

NC = 2
NS = 16
NW = NC * NS
CHUNK = 128


def _deg_body(npad, nch, dst3_hbm, degp_hbm, dst_v, ones_v, zero_v, deg_sh):
    c = lax.axis_index("c")
    s = lax.axis_index("s")
    wid = s * NC + c
    rpt = npad // NS

    for i in range(rpt // 16):
        zero_v[pl.ds(i * 16, 16)] = jnp.zeros((16,), jnp.float32)
    pltpu.sync_copy(zero_v, deg_sh.at[pl.ds(s * rpt, rpt)])

    for i in range(CHUNK // 16):
        ones_v[pl.ds(i * 16, 16)] = jnp.ones((16,), jnp.float32)
    pltpu.sync_copy(dst3_hbm.at[wid], dst_v)
    plsc.subcore_barrier()

    def step(j, _):
        pltpu.sync_copy(ones_v, deg_sh.at[dst_v.at[j]], add=True)
        return _

    lax.fori_loop(0, nch, step, 0)
    plsc.subcore_barrier()
    pltpu.sync_copy(deg_sh.at[pl.ds(s * rpt, rpt)],
                    degp_hbm.at[c].at[pl.ds(s * rpt, rpt)])


NBUF = 2
IR = 8


def _scat_body(npad, nch, y_hbm, idx3_hbm, outp_hbm, idxc, rows_v, *rest):
    gsems = rest[:NBUF]
    ssems = rest[NBUF:2 * NBUF]
    isems = rest[2 * NBUF:2 * NBUF + IR]
    accum_sh = rest[2 * NBUF + IR]
    c = lax.axis_index("c")
    s = lax.axis_index("s")
    wid = s * NC + c
    rpt = npad // NS
    my_idx = idx3_hbm.at[wid]

    pltpu.sync_copy(y_hbm.at[pl.ds(s * rpt, rpt)],
                    accum_sh.at[pl.ds(s * rpt, rpt)])
    plsc.subcore_barrier()

    def idxload(j, u):
        pltpu.async_copy(my_idx.at[j], idxc.at[u], isems[u])

    def wait_idx(u):
        pltpu.make_async_copy(my_idx.at[0], idxc.at[u], isems[u]).wait()

    def gather(j, u, b):
        del j
        pltpu.async_copy(y_hbm.at[idxc.at[u, 0]], rows_v.at[b], gsems[b])

    def wait_gather(b):
        pltpu.make_async_copy(y_hbm.at[idxc.at[0, 0]], rows_v.at[b],
                              gsems[b]).wait()

    def scat(j, u, b):
        del j
        pltpu.async_copy(rows_v.at[b], accum_sh.at[idxc.at[u, 1]], ssems[b],
                         add=True)

    def wait_scat(b):
        pltpu.make_async_copy(rows_v.at[b], accum_sh.at[idxc.at[0, 1]],
                              ssems[b]).wait()

    for u in range(IR):
        idxload(u, u)
    for j in range(NBUF):
        wait_idx(j)
        gather(j, j, j)

    def body(t, _):
        base = t * IR
        for u in range(IR):
            j = base + u
            b = u % NBUF
            wait_gather(b)
            scat(j, u, b)
            wait_scat(b)
            idxload(j + IR, u)
            wait_idx((u + NBUF) % IR)
            gather(j + NBUF, (u + NBUF) % IR, b)
        return _

    lax.fori_loop(0, nch // IR - 1, body, 0)

    base = nch - IR
    for u in range(IR):
        j = base + u
        b = u % NBUF
        wait_gather(b)
        scat(j, u, b)
        wait_scat(b)
        if u + NBUF < IR:
            wait_idx(u + NBUF)
            gather(j + NBUF, u + NBUF, b)
    plsc.subcore_barrier()
    pltpu.sync_copy(accum_sh.at[pl.ds(s * rpt, rpt)],
                    outp_hbm.at[c].at[pl.ds(s * rpt, rpt)])


def _mm_body(x_ref, w_ref, degp_ref, y_ref, dinv_ref):
    deg = degp_ref[0] + degp_ref[1] + 1.0
    dinv = lax.rsqrt(deg)
    xw = jnp.dot(x_ref[...], w_ref[...], preferred_element_type=jnp.float32)
    y_ref[...] = xw * dinv
    dinv_ref[...] = dinv


def _fin_body(outp_ref, y_ref, dinv_ref, b_ref, out_ref):
    acc = outp_ref[0] + outp_ref[1] - y_ref[...]
    out_ref[...] = jnp.maximum(acc * dinv_ref[...] + b_ref[...], 0.0)


def kernel(x, edge_index, W, b):
    N, D = x.shape
    E = edge_index.shape[1]
    NPAD = ((N + NS * 16 - 1) // (NS * 16)) * (NS * 16)
    NPAD = max(NPAD, ((N + 127) // 128) * 128)
    nch = -(-E // (NW * CHUNK))
    nch = -(-nch // IR) * IR
    epad = NW * CHUNK * nch - E

    src = edge_index[0].astype(jnp.int32)
    dst = edge_index[1].astype(jnp.int32)
    pad_idx = N + (jnp.arange(epad, dtype=jnp.int32) % (NPAD - N))
    src3 = jnp.concatenate([src, pad_idx]).reshape(NW, nch, 1, CHUNK)
    dst3 = jnp.concatenate([dst, pad_idx]).reshape(NW, nch, 1, CHUNK)
    idx3 = jnp.concatenate([src3, dst3], axis=2)
    x_pad = jnp.concatenate([x, jnp.zeros((NPAD - N, D), x.dtype)])

    mesh = plsc.VectorSubcoreMesh(core_axis_name="c", subcore_axis_name="s",
                                  num_cores=NC, num_subcores=NS)

    degp = pl.kernel(
        functools.partial(_deg_body, NPAD, nch),
        out_type=jax.ShapeDtypeStruct((NC, NPAD), jnp.float32),
        mesh=mesh,
        scratch_types=[
            pltpu.VMEM((nch, CHUNK), jnp.int32),
            pltpu.VMEM((CHUNK,), jnp.float32),
            pltpu.VMEM((NPAD // NS,), jnp.float32),
            pltpu.VMEM_SHARED((NPAD,), jnp.float32),
        ],
    )(dst3.reshape(NW, nch, CHUNK))

    BR = NPAD // 8
    y_pad, dinv = pl.pallas_call(
        _mm_body,
        grid=(8,),
        in_specs=[
            pl.BlockSpec((BR, D), lambda i: (i, 0)),
            pl.BlockSpec((D, D), lambda i: (0, 0)),
            pl.BlockSpec((NC, BR, 1), lambda i: (0, i, 0)),
        ],
        out_specs=[
            pl.BlockSpec((BR, D), lambda i: (i, 0)),
            pl.BlockSpec((BR, 1), lambda i: (i, 0)),
        ],
        out_shape=[
            jax.ShapeDtypeStruct((NPAD, D), jnp.float32),
            jax.ShapeDtypeStruct((NPAD, 1), jnp.float32),
        ],
    )(x_pad, W, degp.reshape(NC, NPAD, 1))

    outp = pl.kernel(
        functools.partial(_scat_body, NPAD, nch),
        out_type=jax.ShapeDtypeStruct((NC, NPAD, D), jnp.float32),
        mesh=mesh,
        scratch_types=[
            pltpu.VMEM((IR, 2, CHUNK), jnp.int32),
            pltpu.VMEM((NBUF, CHUNK, D), jnp.float32),
        ] + [pltpu.SemaphoreType.DMA] * (2 * NBUF + IR) + [
            pltpu.VMEM_SHARED((NPAD, D), jnp.float32),
        ],
    )(y_pad, idx3)

    RB = 2000
    out = pl.pallas_call(
        _fin_body,
        grid=(N // RB,),
        in_specs=[
            pl.BlockSpec((NC, RB, D), lambda i: (0, i, 0)),
            pl.BlockSpec((RB, D), lambda i: (i, 0)),
            pl.BlockSpec((RB, 1), lambda i: (i, 0)),
            pl.BlockSpec((1, D), lambda i: (0, 0)),
        ],
        out_specs=pl.BlockSpec((RB, D), lambda i: (i, 0)),
        out_shape=jax.ShapeDtypeStruct((N, D), jnp.float32),
    )(outp, y_pad, dinv, b.reshape(1, D))
    return out


# speedup vs baseline: 44.9432x; 1.0515x over previous
import functools

import jax
import jax.numpy as jnp
from jax import lax
from jax.experimental import pallas as pl
from jax.experimental.pallas import tpu as pltpu
from jax.experimental.pallas import tpu_sc as plsc

NC = 2
NS = 16
NW = NC * NS
CHUNK = 128


def _deg_body(npad, nch, dst3_hbm, degp_hbm, dst_v, ones_v, zero_v, deg_sh):
    c = lax.axis_index("c")
    s = lax.axis_index("s")
    wid = s * NC + c
    rpt = npad // NS

    for i in range(rpt // 16):
        zero_v[pl.ds(i * 16, 16)] = jnp.zeros((16,), jnp.float32)
    pltpu.sync_copy(zero_v, deg_sh.at[pl.ds(s * rpt, rpt)])

    for i in range(CHUNK // 16):
        ones_v[pl.ds(i * 16, 16)] = jnp.ones((16,), jnp.float32)
    pltpu.sync_copy(dst3_hbm.at[wid], dst_v)
    plsc.subcore_barrier()

    def step(j, _):
        pltpu.sync_copy(ones_v, deg_sh.at[dst_v.at[j]], add=True)
        return _

    lax.fori_loop(0, nch, step, 0)
    plsc.subcore_barrier()
    pltpu.sync_copy(deg_sh.at[pl.ds(s * rpt, rpt)],
                    degp_hbm.at[c].at[pl.ds(s * rpt, rpt)])


NBUF = 2
IR = 8


def _scat_body(npad, nch, y_hbm, src3_hbm, dst3_hbm, outp_hbm, srcc, dstc,
               rows_v, *rest):
    gsems = rest[:NBUF]
    ssems = rest[NBUF:2 * NBUF]
    isems = rest[2 * NBUF:2 * NBUF + IR]
    accum_sh = rest[2 * NBUF + IR]
    c = lax.axis_index("c")
    s = lax.axis_index("s")
    wid = s * NC + c
    rpt = npad // NS
    my_src = src3_hbm.at[wid]
    my_dst = dst3_hbm.at[wid]

    pltpu.sync_copy(y_hbm.at[pl.ds(s * rpt, rpt)],
                    accum_sh.at[pl.ds(s * rpt, rpt)])
    plsc.subcore_barrier()

    def idxload(j, u):
        pltpu.async_copy(my_src.at[j], srcc.at[u], isems[u])
        pltpu.async_copy(my_dst.at[j], dstc.at[u], isems[u])

    def wait_idx(u):
        pltpu.make_async_copy(my_src.at[0], srcc.at[u], isems[u]).wait()
        pltpu.make_async_copy(my_dst.at[0], dstc.at[u], isems[u]).wait()

    def gather(j, u, b):
        del j
        pltpu.async_copy(y_hbm.at[srcc.at[u]], rows_v.at[b], gsems[b])

    def wait_gather(b):
        pltpu.make_async_copy(y_hbm.at[srcc.at[0]], rows_v.at[b],
                              gsems[b]).wait()

    def scat(j, u, b):
        del j
        pltpu.async_copy(rows_v.at[b], accum_sh.at[dstc.at[u]], ssems[b],
                         add=True)

    def wait_scat(b):
        pltpu.make_async_copy(rows_v.at[b], accum_sh.at[dstc.at[0]],
                              ssems[b]).wait()

    for u in range(IR):
        idxload(u, u)
    for j in range(NBUF):
        wait_idx(j)
        gather(j, j, j)

    def body(t, _):
        base = t * IR
        for u in range(IR):
            j = base + u
            b = u % NBUF
            wait_gather(b)
            scat(j, u, b)
            wait_scat(b)
            idxload(j + IR, u)
            wait_idx((u + NBUF) % IR)
            gather(j + NBUF, (u + NBUF) % IR, b)
        return _

    lax.fori_loop(0, nch // IR - 1, body, 0)

    base = nch - IR
    for u in range(IR):
        j = base + u
        b = u % NBUF
        wait_gather(b)
        scat(j, u, b)
        wait_scat(b)
        if u + NBUF < IR:
            wait_idx(u + NBUF)
            gather(j + NBUF, u + NBUF, b)
    plsc.subcore_barrier()
    pltpu.sync_copy(accum_sh.at[pl.ds(s * rpt, rpt)],
                    outp_hbm.at[c].at[pl.ds(s * rpt, rpt)])


def _xw_body(x_ref, w_ref, xw_ref):
    xw_ref[...] = jnp.dot(x_ref[...], w_ref[...],
                          preferred_element_type=jnp.float32)


def _scale_body(xw_ref, degp_ref, y_ref, dinv_ref):
    deg = degp_ref[0] + degp_ref[1] + 1.0
    dinv = lax.rsqrt(deg)[:, None]
    y_ref[...] = xw_ref[...] * dinv
    dinv_ref[...] = dinv


def _fin_body(outp_ref, y_ref, dinv_ref, b_ref, out_ref):
    acc = outp_ref[0] + outp_ref[1] - y_ref[...]
    out_ref[...] = jnp.maximum(acc * dinv_ref[...] + b_ref[...], 0.0)


def kernel(x, edge_index, W, b):
    N, D = x.shape
    E = edge_index.shape[1]
    NPAD = ((N + NS * 16 - 1) // (NS * 16)) * (NS * 16)
    NPAD = max(NPAD, ((N + 127) // 128) * 128)
    nch = -(-E // (NW * CHUNK))
    nch = -(-nch // IR) * IR
    epad = NW * CHUNK * nch - E

    src = edge_index[0].astype(jnp.int32)
    dst = edge_index[1].astype(jnp.int32)
    pad_idx = N + (jnp.arange(epad, dtype=jnp.int32) % (NPAD - N))
    src3 = jnp.concatenate([src, pad_idx]).reshape(NW, nch, CHUNK)
    dst3 = jnp.concatenate([dst, pad_idx]).reshape(NW, nch, CHUNK)

    mesh = plsc.VectorSubcoreMesh(core_axis_name="c", subcore_axis_name="s",
                                  num_cores=NC, num_subcores=NS)

    degp = pl.kernel(
        functools.partial(_deg_body, NPAD, nch),
        out_type=jax.ShapeDtypeStruct((NC, NPAD), jnp.float32),
        mesh=mesh,
        scratch_types=[
            pltpu.VMEM((nch, CHUNK), jnp.int32),
            pltpu.VMEM((CHUNK,), jnp.float32),
            pltpu.VMEM((NPAD // NS,), jnp.float32),
            pltpu.VMEM_SHARED((NPAD,), jnp.float32),
        ],
    )(dst3)

    BR = NPAD // 8
    xw = pl.pallas_call(
        _xw_body,
        grid=(8,),
        in_specs=[
            pl.BlockSpec((BR, D), lambda i: (i, 0)),
            pl.BlockSpec((D, D), lambda i: (0, 0)),
        ],
        out_specs=pl.BlockSpec((BR, D), lambda i: (i, 0)),
        out_shape=jax.ShapeDtypeStruct((NPAD, D), jnp.float32),
    )(x, W)

    y_pad, dinv = pl.pallas_call(
        _scale_body,
        grid=(8,),
        in_specs=[
            pl.BlockSpec((BR, D), lambda i: (i, 0)),
            pl.BlockSpec((NC, BR), lambda i: (0, i)),
        ],
        out_specs=[
            pl.BlockSpec((BR, D), lambda i: (i, 0)),
            pl.BlockSpec((BR, 1), lambda i: (i, 0)),
        ],
        out_shape=[
            jax.ShapeDtypeStruct((NPAD, D), jnp.float32),
            jax.ShapeDtypeStruct((NPAD, 1), jnp.float32),
        ],
    )(xw, degp)

    outp = pl.kernel(
        functools.partial(_scat_body, NPAD, nch),
        out_type=jax.ShapeDtypeStruct((NC, NPAD, D), jnp.float32),
        mesh=mesh,
        scratch_types=[
            pltpu.VMEM((IR, CHUNK), jnp.int32),
            pltpu.VMEM((IR, CHUNK), jnp.int32),
            pltpu.VMEM((NBUF, CHUNK, D), jnp.float32),
        ] + [pltpu.SemaphoreType.DMA] * (2 * NBUF + IR) + [
            pltpu.VMEM_SHARED((NPAD, D), jnp.float32),
        ],
    )(y_pad, src3, dst3)

    RB = 2000
    out = pl.pallas_call(
        _fin_body,
        grid=(N // RB,),
        in_specs=[
            pl.BlockSpec((NC, RB, D), lambda i: (0, i, 0)),
            pl.BlockSpec((RB, D), lambda i: (i, 0)),
            pl.BlockSpec((RB, 1), lambda i: (i, 0)),
            pl.BlockSpec((1, D), lambda i: (0, 0)),
        ],
        out_specs=pl.BlockSpec((RB, D), lambda i: (i, 0)),
        out_shape=jax.ShapeDtypeStruct((N, D), jnp.float32),
    )(outp, y_pad, dinv, b.reshape(1, D))
    return out
